# baseline (speedup 1.0000x reference)
import jax
import jax.numpy as jnp
from jax import lax
from jax.experimental import pallas as pl
from jax.experimental.pallas import tpu as pltpu
from jax.experimental.pallas import tpu_sc as plsc

B, C, H, W = 2, 96, 256, 256
HO = WO = 512
L = 16
NC, NS = 2, 16
NW = NC * NS
R = 32
IN_W = R * W
OUT_W = 2 * R * WO
BLOCKS_PER_PLANE = H // R
PLANES_PER_WORKER = (B * C) // NW
TPW = PLANES_PER_WORKER * BLOCKS_PER_PLANE
NBUF = 2


def _unpool_body(vals_hbm, idx_hbm, out_hbm,
                 v0, v1, i0, i1, o0, o1,
                 sv0, sv1, si0, si1, so0, so1):
    vb, ib, ob = (v0, v1), (i0, i1), (o0, o1)
    sv, si, so = (sv0, sv1), (si0, si1), (so0, so1)

    wid = lax.axis_index("s") * NC + lax.axis_index("c")
    b_idx = wid >> 4
    c_base = (wid & 15) * PLANES_PER_WORKER

    def in_copy(slot, t):
        p = t >> 3
        blk = t & 7
        src_v = vals_hbm.at[b_idx, c_base + p, pl.ds(blk * R, R), :]
        src_i = idx_hbm.at[b_idx, c_base + p, pl.ds(blk * R, R), :]
        return (pltpu.make_async_copy(src_v, vb[slot], sv[slot]),
                pltpu.make_async_copy(src_i, ib[slot], si[slot]))

    def start_in(slot, t):
        for c in in_copy(slot, t):
            c.start()

    def wait_in(slot, t):
        for c in in_copy(slot, t):
            c.wait()

    def out_copy(slot, t):
        p = t >> 3
        blk = t & 7
        dst = out_hbm.at[b_idx, c_base + p, pl.ds(blk * 2 * R, 2 * R), :]
        return pltpu.make_async_copy(ob[slot], dst, so[slot])

    for slot in range(NBUF):
        start_in(slot, slot)

    def outer(step, carry):
        for slot in range(NBUF):
            t = step * NBUF + slot
            wait_in(slot, t)

            @pl.when(t >= NBUF)
            def _():
                out_copy(slot, t - NBUF).wait()

            blk = t & 7
            row0 = blk * 2 * R
            vbuf, ibuf, obuf = vb[slot], ib[slot], ob[slot]
            zero = jnp.zeros((L,), jnp.float32)

            @plsc.parallel_loop(0, R, unroll=1)
            def inner(i_row):
                or0 = 2 * i_row
                or1 = or0 + 1
                for jj in range(0, WO, L):
                    obuf[or0, pl.ds(jj, L)] = zero
                    obuf[or1, pl.ds(jj, L)] = zero
                for jv in range(0, W, L):
                    v = vbuf[i_row, pl.ds(jv, L)]
                    ix = ibuf[i_row, pl.ds(jv, L)]
                    lr = (ix >> 9) - row0
                    lc = ix & 511
                    plsc.store_scatter(obuf, [lr, lc], v)

            out_copy(slot, t).start()

            @pl.when(t + NBUF < TPW)
            def _():
                start_in(slot, t + NBUF)
        return carry

    lax.fori_loop(0, TPW // NBUF, outer, 0)
    for slot in range(NBUF):
        out_copy(slot, TPW - NBUF + slot).wait()


_unpool_call = pl.kernel(
    _unpool_body,
    out_type=jax.ShapeDtypeStruct((B, C, HO, WO), jnp.float32),
    mesh=plsc.VectorSubcoreMesh(
        core_axis_name="c", subcore_axis_name="s",
        num_cores=NC, num_subcores=NS),
    compiler_params=pltpu.CompilerParams(
        needs_layout_passes=False, use_tc_tiling_on_sc=True,
        skip_device_barrier=True),
    scratch_types=[
        pltpu.VMEM((R, W), jnp.float32),
        pltpu.VMEM((R, W), jnp.float32),
        pltpu.VMEM((R, W), jnp.int32),
        pltpu.VMEM((R, W), jnp.int32),
        pltpu.VMEM((2 * R, WO), jnp.float32),
        pltpu.VMEM((2 * R, WO), jnp.float32),
        pltpu.SemaphoreType.DMA,
        pltpu.SemaphoreType.DMA,
        pltpu.SemaphoreType.DMA,
        pltpu.SemaphoreType.DMA,
        pltpu.SemaphoreType.DMA,
        pltpu.SemaphoreType.DMA,
    ],
)


def kernel(inputs, indices):
    return _unpool_call(inputs, indices)


# speedup vs baseline: 1.0051x; 1.0051x over previous
import jax
import jax.numpy as jnp
from jax import lax
from jax.experimental import pallas as pl
from jax.experimental.pallas import tpu as pltpu
from jax.experimental.pallas import tpu_sc as plsc

B, C, H, W = 2, 96, 256, 256
HO = WO = 512
L = 16
NC, NS = 2, 16
NW = NC * NS
R = 32
IN_W = R * W
OUT_W = 2 * R * WO
BLOCKS_PER_PLANE = H // R
PLANES_PER_WORKER = (B * C) // NW
TPW = PLANES_PER_WORKER * BLOCKS_PER_PLANE
NBUF = 2


def _unpool_body(vals_hbm, idx_hbm, out_hbm,
                 v0, v1, i0, i1, o0, o1,
                 sv0, sv1, si0, si1, so0, so1):
    vb, ib, ob = (v0, v1), (i0, i1), (o0, o1)
    sv, si, so = (sv0, sv1), (si0, si1), (so0, so1)

    wid = lax.axis_index("s") * NC + lax.axis_index("c")
    b_idx = wid >> 4
    c_base = (wid & 15) * PLANES_PER_WORKER

    def in_copy(slot, t):
        p = t >> 3
        blk = t & 7
        src_v = vals_hbm.at[b_idx, c_base + p, pl.ds(blk * R, R), :]
        src_i = idx_hbm.at[b_idx, c_base + p, pl.ds(blk * R, R), :]
        return (pltpu.make_async_copy(src_v, vb[slot], sv[slot]),
                pltpu.make_async_copy(src_i, ib[slot], si[slot]))

    def start_in(slot, t):
        for c in in_copy(slot, t):
            c.start()

    def wait_in(slot, t):
        for c in in_copy(slot, t):
            c.wait()

    def out_copy(slot, t):
        p = t >> 3
        blk = t & 7
        dst = out_hbm.at[b_idx, c_base + p, pl.ds(blk * 2 * R, 2 * R), :]
        return pltpu.make_async_copy(ob[slot], dst, so[slot])

    for slot in range(NBUF):
        start_in(slot, slot)

    def outer(step, carry):
        for slot in range(NBUF):
            t = step * NBUF + slot
            wait_in(slot, t)

            @pl.when(t >= NBUF)
            def _():
                out_copy(slot, t - NBUF).wait()

            blk = t & 7
            row0 = blk * 2 * R
            vbuf, ibuf, obuf = vb[slot], ib[slot], ob[slot]
            zero = jnp.zeros((L,), jnp.float32)

            @plsc.parallel_loop(0, R, unroll=1)
            def inner(i_row):
                or0 = 2 * i_row
                or1 = or0 + 1
                for jj in range(0, WO, L):
                    obuf[or0, pl.ds(jj, L)] = zero
                    obuf[or1, pl.ds(jj, L)] = zero
                for jv in range(0, W, L):
                    v = vbuf[i_row, pl.ds(jv, L)]
                    ix = ibuf[i_row, pl.ds(jv, L)]
                    lr = (ix >> 9) - row0
                    lc = ix & 511
                    plsc.store_scatter(obuf, [lr, lc], v)

            out_copy(slot, t).start()

            @pl.when(t + NBUF < TPW)
            def _():
                start_in(slot, t + NBUF)
        return carry

    lax.fori_loop(0, TPW // NBUF, outer, 0)
    for slot in range(NBUF):
        out_copy(slot, TPW - NBUF + slot).wait()


_unpool_call = pl.kernel(
    _unpool_body,
    out_type=jax.ShapeDtypeStruct((B, C, HO, WO), jnp.float32),
    mesh=plsc.VectorSubcoreMesh(
        core_axis_name="c", subcore_axis_name="s",
        num_cores=NC, num_subcores=NS),
    compiler_params=pltpu.CompilerParams(
        needs_layout_passes=False, use_tc_tiling_on_sc=True),
    scratch_types=[
        pltpu.VMEM((R, W), jnp.float32),
        pltpu.VMEM((R, W), jnp.float32),
        pltpu.VMEM((R, W), jnp.int32),
        pltpu.VMEM((R, W), jnp.int32),
        pltpu.VMEM((2 * R, WO), jnp.float32),
        pltpu.VMEM((2 * R, WO), jnp.float32),
        pltpu.SemaphoreType.DMA,
        pltpu.SemaphoreType.DMA,
        pltpu.SemaphoreType.DMA,
        pltpu.SemaphoreType.DMA,
        pltpu.SemaphoreType.DMA,
        pltpu.SemaphoreType.DMA,
    ],
)


def kernel(inputs, indices):
    return _unpool_call(inputs, indices)
